# baseline (device time: 78019 ns/iter reference)
import jax
import jax.numpy as jnp
from jax import lax
from jax.experimental import pallas as pl
from jax.experimental.pallas import tpu as pltpu

N_DEV = 4


def kernel(x, w_mat):
    m_total, k_per = x.shape
    _, n = w_mat.shape
    m_per = m_total // N_DEV
    half = m_per // 2
    oblk = m_per // 4

    def body(x_hbm, w_hbm, out_hbm, x_stage, send_bf, recv_buf,
             w_stage, w_bf, y_acc, amax_buf, xcopy_sems, wcopy_sem,
             ocopy_sems, send_sems, recv_sems, amax_send_sems,
             amax_recv_sems):
        my = lax.axis_index("i")
        right = (my + 1) % N_DEV
        left = (my - 1) % N_DEV
        opp = (my + 2) % N_DEV


        def x_chunk_copy(tgt, stage_slot):
            return pltpu.make_async_copy(
                x_hbm.at[pl.ds(tgt * m_per, m_per), :],
                x_stage.at[stage_slot],
                xcopy_sems.at[stage_slot],
            )

        def chunk_rdma(tgt, slot, sem_idx, row0, nrows):
            return pltpu.make_async_remote_copy(
                src_ref=send_bf.at[slot, pl.ds(row0, nrows), :],
                dst_ref=recv_buf.at[slot, pl.ds(row0, nrows), :],
                send_sem=send_sems.at[sem_idx],
                recv_sem=recv_sems.at[sem_idx],
                device_id=(tgt,),
                device_id_type=pl.DeviceIdType.MESH,
            )

        order = ((right, 0), (left, 1), (opp, 2), (my, 3))
        copies = [x_chunk_copy(order[0][0], 0), x_chunk_copy(order[1][0], 1)]
        copies[0].start()
        copies[1].start()

        barrier = pltpu.get_barrier_semaphore()
        for peer in (left, right, opp):
            pl.semaphore_signal(barrier, inc=1, device_id=(peer,),
                                device_id_type=pl.DeviceIdType.MESH)
        pl.semaphore_wait(barrier, 3)

        rdmas = {}
        for idx, (tgt, slot) in enumerate(order):
            copies[idx].wait()
            send_bf[slot] = x_stage[idx % 2].astype(jnp.bfloat16)
            if slot < 2:
                rdmas[slot] = chunk_rdma(tgt, slot, slot, 0, m_per)
                rdmas[slot].start()
            elif slot == 2:
                rdmas[2] = chunk_rdma(tgt, 2, 2, 0, half)
                rdmas[3] = chunk_rdma(tgt, 2, 3, half, half)
                rdmas[2].start()
                rdmas[3].start()
            if idx + 2 < len(order):
                copies.append(x_chunk_copy(order[idx + 2][0], idx % 2))
                copies[idx + 2].start()

        def w_block_copy(src_dev):
            return pltpu.make_async_copy(
                w_hbm.at[pl.ds(src_dev * m_per, m_per), :],
                w_stage,
                wcopy_sem,
            )

        wcp = w_block_copy(my)
        wcp.start()
        wcp.wait()
        w_bf[...] = w_stage[...].astype(jnp.bfloat16)
        wcp = w_block_copy(left)
        wcp.start()

        y_acc[...] = jnp.dot(
            send_bf[3], w_bf[...], preferred_element_type=jnp.float32)

        for slot, nxt in ((0, right), (1, opp)):
            wcp.wait()
            w_bf[...] = w_stage[...].astype(jnp.bfloat16)
            wcp = w_block_copy(nxt)
            wcp.start()
            rdmas[slot].wait_recv()
            y_acc[...] += jnp.dot(
                recv_buf[slot], w_bf[...], preferred_element_type=jnp.float32)

        wcp.wait()
        w_bf[...] = w_stage[...].astype(jnp.bfloat16)
        rdmas[2].wait_recv()
        y_acc[0:half, :] += jnp.dot(
            recv_buf[2, 0:half, :], w_bf[...],
            preferred_element_type=jnp.float32)
        amax_lo = jnp.max(jnp.maximum(y_acc[0:half, :], 0.0))
        rdmas[3].wait_recv()
        y_acc[half:, :] += jnp.dot(
            recv_buf[2, half:, :], w_bf[...],
            preferred_element_type=jnp.float32)
        amax_hi = jnp.max(jnp.maximum(y_acc[half:, :], 0.0))
        for sem_idx in (0, 1, 2, 3):
            rdmas[sem_idx].wait_send()

        local_amax = jnp.maximum(amax_lo, amax_hi)
        amax_buf[3] = jnp.full((8, 128), local_amax, jnp.float32)
        amax_rdmas = []
        for tgt, slot in ((right, 0), (left, 1), (opp, 2)):
            r = pltpu.make_async_remote_copy(
                src_ref=amax_buf.at[3],
                dst_ref=amax_buf.at[slot],
                send_sem=amax_send_sems.at[slot],
                recv_sem=amax_recv_sems.at[slot],
                device_id=(tgt,),
                device_id_type=pl.DeviceIdType.MESH,
            )
            r.start()
            amax_rdmas.append(r)
        for r in amax_rdmas:
            r.wait_recv()
        for r in amax_rdmas:
            r.wait_send()

        gmax = jnp.max(amax_buf[...])
        scale = gmax * (1.0 / 448.0)
        inv_scale = 448.0 / gmax
        ocopies = []
        for b in range(m_per // oblk):
            r0, r1 = b * oblk, (b + 1) * oblk
            yb = jnp.maximum(y_acc[r0:r1, :], 0.0)
            qb = jnp.minimum(yb * inv_scale, 448.0).astype(jnp.float8_e4m3fn)
            y_acc[r0:r1, :] = qb.astype(jnp.float32) * scale
            cp = pltpu.make_async_copy(
                y_acc.at[pl.ds(r0, oblk), :],
                out_hbm.at[pl.ds(r0, oblk), :],
                ocopy_sems.at[b],
            )
            cp.start()
            ocopies.append(cp)
        for cp in ocopies:
            cp.wait()

    return pl.pallas_call(
        body,
        out_shape=jax.ShapeDtypeStruct((m_per, n), jnp.float32),
        in_specs=[
            pl.BlockSpec(memory_space=pl.ANY),
            pl.BlockSpec(memory_space=pl.ANY),
        ],
        out_specs=pl.BlockSpec(memory_space=pl.ANY),
        scratch_shapes=[
            pltpu.VMEM((2, m_per, k_per), jnp.float32),
            pltpu.VMEM((4, m_per, k_per), jnp.bfloat16),
            pltpu.VMEM((3, m_per, k_per), jnp.bfloat16),
            pltpu.VMEM((m_per, n), jnp.float32),
            pltpu.VMEM((m_per, n), jnp.bfloat16),
            pltpu.VMEM((m_per, n), jnp.float32),
            pltpu.VMEM((4, 8, 128), jnp.float32),
            pltpu.SemaphoreType.DMA((2,)),
            pltpu.SemaphoreType.DMA,
            pltpu.SemaphoreType.DMA((4,)),
            pltpu.SemaphoreType.DMA((4,)),
            pltpu.SemaphoreType.DMA((4,)),
            pltpu.SemaphoreType.DMA((3,)),
            pltpu.SemaphoreType.DMA((3,)),
        ],
        compiler_params=pltpu.CompilerParams(
            collective_id=0,
            vmem_limit_bytes=60 * 1024 * 1024,
        ),
    )(x, w_mat)


# device time: 71650 ns/iter; 1.0889x vs baseline; 1.0889x over previous
import jax
import jax.numpy as jnp
from jax import lax
from jax.experimental import pallas as pl
from jax.experimental.pallas import tpu as pltpu

N_DEV = 4


def kernel(x, w_mat):
    m_total, k_per = x.shape
    _, n = w_mat.shape
    m_per = m_total // N_DEV
    half = m_per // 2
    oblk = m_per // 4

    def body(x_hbm, w_hbm, out_hbm, x_stage, send_bf, recv_buf,
             w_stage, w_bf, y_acc, amax_buf, xcopy_sems, wcopy_sem,
             ocopy_sems, send_sems, recv_sems, amax_send_sems,
             amax_recv_sems):
        my = lax.axis_index("i")
        right = (my + 1) % N_DEV
        left = (my - 1) % N_DEV
        opp = (my + 2) % N_DEV


        def x_chunk_copy(tgt, stage_slot):
            return pltpu.make_async_copy(
                x_hbm.at[pl.ds(tgt * m_per, m_per), :],
                x_stage.at[stage_slot],
                xcopy_sems.at[stage_slot],
            )

        def chunk_rdma(tgt, slot, sem_idx, row0, nrows):
            return pltpu.make_async_remote_copy(
                src_ref=send_bf.at[slot, pl.ds(row0, nrows), :],
                dst_ref=recv_buf.at[slot, pl.ds(row0, nrows), :],
                send_sem=send_sems.at[sem_idx],
                recv_sem=recv_sems.at[sem_idx],
                device_id=(tgt,),
                device_id_type=pl.DeviceIdType.MESH,
            )

        barrier = pltpu.get_barrier_semaphore()
        for peer in (left, right, opp):
            pl.semaphore_signal(barrier, inc=1, device_id=(peer,),
                                device_id_type=pl.DeviceIdType.MESH)
        pl.semaphore_wait(barrier, 3)

        order = ((right, 0), (left, 1), (opp, 2), (my, 3))
        copies = [x_chunk_copy(order[0][0], 0), x_chunk_copy(order[1][0], 1)]
        copies[0].start()
        copies[1].start()
        rdmas = {}
        for idx, (tgt, slot) in enumerate(order):
            copies[idx].wait()
            send_bf[slot] = x_stage[idx % 2].astype(jnp.bfloat16)
            if slot < 2:
                rdmas[slot] = chunk_rdma(tgt, slot, slot, 0, m_per)
                rdmas[slot].start()
            elif slot == 2:
                rdmas[2] = chunk_rdma(tgt, 2, 2, 0, half)
                rdmas[3] = chunk_rdma(tgt, 2, 3, half, half)
                rdmas[2].start()
                rdmas[3].start()
            if idx + 2 < len(order):
                copies.append(x_chunk_copy(order[idx + 2][0], idx % 2))
                copies[idx + 2].start()

        def w_block_copy(src_dev):
            return pltpu.make_async_copy(
                w_hbm.at[pl.ds(src_dev * m_per, m_per), :],
                w_stage,
                wcopy_sem,
            )

        wcp = w_block_copy(my)
        wcp.start()
        wcp.wait()
        w_bf[...] = w_stage[...].astype(jnp.bfloat16)
        wcp = w_block_copy(left)
        wcp.start()

        y_acc[...] = jnp.dot(
            send_bf[3], w_bf[...], preferred_element_type=jnp.float32)

        for slot, nxt in ((0, right), (1, opp)):
            wcp.wait()
            w_bf[...] = w_stage[...].astype(jnp.bfloat16)
            wcp = w_block_copy(nxt)
            wcp.start()
            rdmas[slot].wait_recv()
            y_acc[...] += jnp.dot(
                recv_buf[slot], w_bf[...], preferred_element_type=jnp.float32)

        wcp.wait()
        w_bf[...] = w_stage[...].astype(jnp.bfloat16)
        rdmas[2].wait_recv()
        y_acc[0:half, :] += jnp.dot(
            recv_buf[2, 0:half, :], w_bf[...],
            preferred_element_type=jnp.float32)
        amax_lo = jnp.max(jnp.maximum(y_acc[0:half, :], 0.0))
        rdmas[3].wait_recv()
        y_acc[half:, :] += jnp.dot(
            recv_buf[2, half:, :], w_bf[...],
            preferred_element_type=jnp.float32)
        amax_hi = jnp.max(jnp.maximum(y_acc[half:, :], 0.0))
        for sem_idx in (0, 1, 2, 3):
            rdmas[sem_idx].wait_send()

        local_amax = jnp.maximum(amax_lo, amax_hi)
        amax_buf[3] = jnp.full((8, 128), local_amax, jnp.float32)
        amax_rdmas = []
        for tgt, slot in ((right, 0), (left, 1), (opp, 2)):
            r = pltpu.make_async_remote_copy(
                src_ref=amax_buf.at[3],
                dst_ref=amax_buf.at[slot],
                send_sem=amax_send_sems.at[slot],
                recv_sem=amax_recv_sems.at[slot],
                device_id=(tgt,),
                device_id_type=pl.DeviceIdType.MESH,
            )
            r.start()
            amax_rdmas.append(r)
        for r in amax_rdmas:
            r.wait_recv()
        for r in amax_rdmas:
            r.wait_send()

        gmax = jnp.max(amax_buf[...])
        scale = gmax * (1.0 / 448.0)
        inv_scale = 448.0 / gmax
        ocopies = []
        for b in range(m_per // oblk):
            r0, r1 = b * oblk, (b + 1) * oblk
            yb = jnp.maximum(y_acc[r0:r1, :], 0.0)
            qb = jnp.minimum(yb * inv_scale, 448.0).astype(jnp.float8_e4m3fn)
            y_acc[r0:r1, :] = qb.astype(jnp.float32) * scale
            cp = pltpu.make_async_copy(
                y_acc.at[pl.ds(r0, oblk), :],
                out_hbm.at[pl.ds(r0, oblk), :],
                ocopy_sems.at[b],
            )
            cp.start()
            ocopies.append(cp)
        for cp in ocopies:
            cp.wait()

    return pl.pallas_call(
        body,
        out_shape=jax.ShapeDtypeStruct((m_per, n), jnp.float32),
        in_specs=[
            pl.BlockSpec(memory_space=pl.ANY),
            pl.BlockSpec(memory_space=pl.ANY),
        ],
        out_specs=pl.BlockSpec(memory_space=pl.ANY),
        scratch_shapes=[
            pltpu.VMEM((2, m_per, k_per), jnp.float32),
            pltpu.VMEM((4, m_per, k_per), jnp.bfloat16),
            pltpu.VMEM((3, m_per, k_per), jnp.bfloat16),
            pltpu.VMEM((m_per, n), jnp.float32),
            pltpu.VMEM((m_per, n), jnp.bfloat16),
            pltpu.VMEM((m_per, n), jnp.float32),
            pltpu.VMEM((4, 8, 128), jnp.float32),
            pltpu.SemaphoreType.DMA((2,)),
            pltpu.SemaphoreType.DMA,
            pltpu.SemaphoreType.DMA((4,)),
            pltpu.SemaphoreType.DMA((4,)),
            pltpu.SemaphoreType.DMA((4,)),
            pltpu.SemaphoreType.DMA((3,)),
            pltpu.SemaphoreType.DMA((3,)),
        ],
        compiler_params=pltpu.CompilerParams(
            collective_id=0,
            vmem_limit_bytes=60 * 1024 * 1024,
        ),
    )(x, w_mat)
